# baseline (device time: 220807 ns/iter reference)
import jax
import jax.numpy as jnp
from jax import lax
from jax.experimental import pallas as pl
from jax.experimental.pallas import tpu as pltpu

N_DEV = 8
SQ_PER = 256
QBLK = 64
HQ_PER = 8
DH = 128
SKV = 4096
DM = 1024
SCALE = 0.08838834764831843
F32 = jnp.float32
BF16 = jnp.bfloat16


def kernel(x, Wq, K_ext, V_ext, Wo):
    def body(x_ref, wq_ref, k_hbm, v_hbm, wo_ref, out_ref,
             xg, acc, kbuf, vbuf, qbuf, sendbuf, recvbuf,
             ag_send, ag_recv, rs_send, rs_recv, kv_sems):
        my = lax.axis_index("i")
        left = lax.rem(my - 1 + N_DEV, N_DEV)
        right = lax.rem(my + 1, N_DEV)
        h0 = my * HQ_PER

        def kv_dma(h, slot):
            copies = []
            for c in range(3):
                for i, kbk in enumerate(range(c, SKV // QBLK, 3)):
                    copies.append(pltpu.make_async_copy(
                        k_hbm.at[0, pl.ds(kbk * QBLK, QBLK), h0 + h, :],
                        kbuf.at[slot, c, pl.ds(i * QBLK, QBLK)],
                        kv_sems.at[slot, 0]))
                    copies.append(pltpu.make_async_copy(
                        v_hbm.at[0, pl.ds(kbk * QBLK, QBLK), h0 + h, :],
                        vbuf.at[slot, c, pl.ds(i * QBLK, QBLK)],
                        kv_sems.at[slot, 1]))
            return copies

        for slot in range(2):
            for c in (1, 2):
                kbuf[slot, c, pl.ds(21 * QBLK, QBLK)] = jnp.zeros(
                    (QBLK, DH), F32)
                vbuf[slot, c, pl.ds(21 * QBLK, QBLK)] = jnp.zeros(
                    (QBLK, DH), F32)

        for cpy in kv_dma(0, 0):
            cpy.start()

        xg[pl.ds(my, 1)] = x_ref[:].astype(BF16)

        barrier = pltpu.get_barrier_semaphore()
        for nbr in (left, right):
            pl.semaphore_signal(barrier, inc=1, device_id=(nbr,),
                                device_id_type=pl.DeviceIdType.MESH)
        pl.semaphore_wait(barrier, 2)

        wqb = wq_ref[:].astype(BF16)

        def rs_desc(t):
            return pltpu.make_async_remote_copy(
                src_ref=sendbuf.at[t % 2],
                dst_ref=recvbuf.at[t],
                send_sem=rs_send.at[t],
                recv_sem=rs_recv.at[t],
                device_id=(right,),
                device_id_type=pl.DeviceIdType.MESH,
            )

        for s in range(N_DEV):
            o = lax.rem(my - s + N_DEV, N_DEV)
            if s < N_DEV - 1:
                ag = pltpu.make_async_remote_copy(
                    src_ref=xg.at[pl.ds(o, 1)],
                    dst_ref=xg.at[pl.ds(o, 1)],
                    send_sem=ag_send.at[s],
                    recv_sem=ag_recv.at[s],
                    device_id=(right,),
                    device_id_type=pl.DeviceIdType.MESH,
                )
                ag.start()

            qbuf[:] = jnp.dot(xg[pl.ds(o, 1)][0], wqb,
                              preferred_element_type=F32).astype(BF16)
            acc[pl.ds(o, 1)] = jnp.zeros((1, SQ_PER, DM), F32)

            def hstep(h, carry, s=s, o=o):
                slot = lax.rem(h, 2)
                for cpy in kv_dma(h, slot):
                    cpy.wait()
                if s < N_DEV - 1:
                    hn = lax.rem(h + 1, HQ_PER)
                    for cpy in kv_dma(hn, 1 - slot):
                        cpy.start()
                else:
                    @pl.when(h < HQ_PER - 1)
                    def _():
                        for cpy in kv_dma(h + 1, 1 - slot):
                            cpy.start()

                ctx_rows = []
                for lqb in range(SQ_PER // QBLK):
                    qb = o * (SQ_PER // QBLK) + lqb
                    r = lax.rem(qb, 3)
                    g = lax.rem(3 - r, 3)
                    dpos = (qb // 3) * QBLK
                    qs = qbuf[pl.ds(lqb * QBLK, QBLK),
                              pl.ds(h * DH, DH)]
                    kmain = kbuf[slot, g].astype(BF16)
                    smain = lax.dot_general(
                        qs, kmain, (((1,), (1,)), ((), ())),
                        preferred_element_type=F32) * SCALE
                    wmain = jnp.exp(smain)
                    kex = jnp.concatenate(
                        [kbuf[slot, 0, pl.ds(0, QBLK)],
                         kbuf[slot, r, pl.ds(dpos, QBLK)]], axis=0
                    ).astype(BF16)
                    sex = lax.dot_general(
                        qs, kex, (((1,), (1,)), ((), ())),
                        preferred_element_type=F32) * SCALE
                    fac = jnp.where(r == 0, 0.0, 1.0).astype(F32)
                    wex = jnp.exp(sex) * fac
                    padc = jnp.where(g == 0, 0.0, float(QBLK)).astype(F32)
                    wsum = (jnp.sum(wmain, axis=1, keepdims=True)
                            + jnp.sum(wex, axis=1, keepdims=True) - padc)
                    vmain = vbuf[slot, g].astype(BF16)
                    vex = jnp.concatenate(
                        [vbuf[slot, 0, pl.ds(0, QBLK)],
                         vbuf[slot, r, pl.ds(dpos, QBLK)]], axis=0
                    ).astype(BF16)
                    ctx_b = (jnp.dot(wmain.astype(BF16), vmain,
                                     preferred_element_type=F32)
                             + jnp.dot(wex.astype(BF16), vex,
                                       preferred_element_type=F32)) / wsum
                    ctx_rows.append(ctx_b)
                ctx_h = jnp.concatenate(ctx_rows, axis=0)
                woh = wo_ref[pl.ds(h * DH, DH), :].astype(BF16)
                part = jnp.dot(ctx_h.astype(BF16), woh,
                               preferred_element_type=F32)
                acc[pl.ds(o, 1)] = acc[pl.ds(o, 1)] + part[None]
                return carry

            lax.fori_loop(0, HQ_PER, hstep, 0)

            if s >= 1:
                t = s - 1
                if t >= 1:
                    rs_desc(t - 1).wait_recv()
                if t >= 2:
                    rs_desc(t - 2).wait_send()
                data = acc[pl.ds(o, 1)]
                if t > 0:
                    data = data + recvbuf[pl.ds(t - 1, 1)]
                sendbuf[pl.ds(t % 2, 1)] = data
                rs_desc(t).start()

            if s < N_DEV - 1:
                ag.wait()

        rs_desc(N_DEV - 3).wait_send()
        last = rs_desc(N_DEV - 2)
        last.wait_send()
        last.wait_recv()
        out_ref[:] = acc[pl.ds(my, 1)] + recvbuf[pl.ds(N_DEV - 2, 1)]

    return pl.pallas_call(
        body,
        out_shape=jax.ShapeDtypeStruct((1, SQ_PER, DM), F32),
        in_specs=[
            pl.BlockSpec(memory_space=pltpu.VMEM),
            pl.BlockSpec(memory_space=pltpu.VMEM),
            pl.BlockSpec(memory_space=pl.ANY),
            pl.BlockSpec(memory_space=pl.ANY),
            pl.BlockSpec(memory_space=pltpu.VMEM),
        ],
        out_specs=pl.BlockSpec(memory_space=pltpu.VMEM),
        scratch_shapes=[
            pltpu.VMEM((N_DEV, SQ_PER, DM), BF16),
            pltpu.VMEM((N_DEV, SQ_PER, DM), F32),
            pltpu.VMEM((2, 3, 22 * QBLK, DH), F32),
            pltpu.VMEM((2, 3, 22 * QBLK, DH), F32),
            pltpu.VMEM((SQ_PER, DM), BF16),
            pltpu.VMEM((2, SQ_PER, DM), F32),
            pltpu.VMEM((N_DEV - 1, SQ_PER, DM), F32),
            pltpu.SemaphoreType.DMA((N_DEV - 1,)),
            pltpu.SemaphoreType.DMA((N_DEV - 1,)),
            pltpu.SemaphoreType.DMA((N_DEV - 1,)),
            pltpu.SemaphoreType.DMA((N_DEV - 1,)),
            pltpu.SemaphoreType.DMA((2, 2)),
        ],
        compiler_params=pltpu.CompilerParams(
            collective_id=0,
            vmem_limit_bytes=60 * 1024 * 1024,
        ),
    )(x, Wq, K_ext, V_ext, Wo)


# device time: 164273 ns/iter; 1.3441x vs baseline; 1.3441x over previous
import jax
import jax.numpy as jnp
from jax import lax
from jax.experimental import pallas as pl
from jax.experimental.pallas import tpu as pltpu

N_DEV = 8
SQ_PER = 256
QBLK = 64
HQ_PER = 8
DH = 128
SKV = 4096
DM = 1024
SCALE = 0.08838834764831843
F32 = jnp.float32
BF16 = jnp.bfloat16


def kernel(x, Wq, K_ext, V_ext, Wo):
    def body(x_ref, wq_ref, k_hbm, v_hbm, wo_ref, out_ref,
             xg, acc, kbuf, vbuf, kres, vres, qbuf, sendbuf, recvbuf,
             ag_send, ag_recv, rs_send, rs_recv, kv_sems):
        my = lax.axis_index("i")
        left = lax.rem(my - 1 + N_DEV, N_DEV)
        right = lax.rem(my + 1, N_DEV)
        h0 = my * HQ_PER

        def kv_dma(h, slot):
            copies = []
            for c in range(3):
                for i, kbk in enumerate(range(c, SKV // QBLK, 3)):
                    copies.append(pltpu.make_async_copy(
                        k_hbm.at[0, pl.ds(kbk * QBLK, QBLK), h0 + h, :],
                        kbuf.at[slot, c, pl.ds(i * QBLK, QBLK)],
                        kv_sems.at[slot, 0]))
                    copies.append(pltpu.make_async_copy(
                        v_hbm.at[0, pl.ds(kbk * QBLK, QBLK), h0 + h, :],
                        vbuf.at[slot, c, pl.ds(i * QBLK, QBLK)],
                        kv_sems.at[slot, 1]))
            return copies

        for slot in range(2):
            for c in (1, 2):
                kbuf[slot, c, pl.ds(21 * QBLK, QBLK)] = jnp.zeros(
                    (QBLK, DH), F32)
                vbuf[slot, c, pl.ds(21 * QBLK, QBLK)] = jnp.zeros(
                    (QBLK, DH), F32)

        for cpy in kv_dma(0, 0):
            cpy.start()

        xg[pl.ds(my, 1)] = x_ref[:].astype(BF16)

        barrier = pltpu.get_barrier_semaphore()
        for nbr in (left, right):
            pl.semaphore_signal(barrier, inc=1, device_id=(nbr,),
                                device_id_type=pl.DeviceIdType.MESH)
        pl.semaphore_wait(barrier, 2)

        wqb = wq_ref[:].astype(BF16)

        def rs_desc(t):
            return pltpu.make_async_remote_copy(
                src_ref=sendbuf.at[t % 2],
                dst_ref=recvbuf.at[t],
                send_sem=rs_send.at[t],
                recv_sem=rs_recv.at[t],
                device_id=(right,),
                device_id_type=pl.DeviceIdType.MESH,
            )

        for s in range(N_DEV):
            o = lax.rem(my - s + N_DEV, N_DEV)
            if s < N_DEV - 1:
                ag = pltpu.make_async_remote_copy(
                    src_ref=xg.at[pl.ds(o, 1)],
                    dst_ref=xg.at[pl.ds(o, 1)],
                    send_sem=ag_send.at[s],
                    recv_sem=ag_recv.at[s],
                    device_id=(right,),
                    device_id_type=pl.DeviceIdType.MESH,
                )
                ag.start()

            qbuf[:] = jnp.dot(xg[pl.ds(o, 1)][0], wqb,
                              preferred_element_type=F32).astype(BF16)
            acc[pl.ds(o, 1)] = jnp.zeros((1, SQ_PER, DM), F32)

            def hstep(h, carry, s=s, o=o):
                if s == 0:
                    slot = lax.rem(h, 2)
                    for cpy in kv_dma(h, slot):
                        cpy.wait()
                    @pl.when(h < HQ_PER - 1)
                    def _():
                        for cpy in kv_dma(h + 1, 1 - slot):
                            cpy.start()
                    kres[h] = kbuf[slot].astype(BF16)
                    vres[h] = vbuf[slot].astype(BF16)

                ctx_rows = []
                for lqb in range(SQ_PER // QBLK):
                    qb = o * (SQ_PER // QBLK) + lqb
                    r = lax.rem(qb, 3)
                    g = lax.rem(3 - r, 3)
                    dpos = (qb // 3) * QBLK
                    qs = qbuf[pl.ds(lqb * QBLK, QBLK),
                              pl.ds(h * DH, DH)]
                    kmain = kres[h, g]
                    smain = lax.dot_general(
                        qs, kmain, (((1,), (1,)), ((), ())),
                        preferred_element_type=F32) * SCALE
                    wmain = jnp.exp(smain)
                    kex = jnp.concatenate(
                        [kres[h, 0, pl.ds(0, QBLK)],
                         kres[h, r, pl.ds(dpos, QBLK)]], axis=0)
                    sex = lax.dot_general(
                        qs, kex, (((1,), (1,)), ((), ())),
                        preferred_element_type=F32) * SCALE
                    fac = jnp.where(r == 0, 0.0, 1.0).astype(F32)
                    wex = jnp.exp(sex) * fac
                    padc = jnp.where(g == 0, 0.0, float(QBLK)).astype(F32)
                    wsum = (jnp.sum(wmain, axis=1, keepdims=True)
                            + jnp.sum(wex, axis=1, keepdims=True) - padc)
                    vmain = vres[h, g]
                    vex = jnp.concatenate(
                        [vres[h, 0, pl.ds(0, QBLK)],
                         vres[h, r, pl.ds(dpos, QBLK)]], axis=0)
                    ctx_b = (jnp.dot(wmain.astype(BF16), vmain,
                                     preferred_element_type=F32)
                             + jnp.dot(wex.astype(BF16), vex,
                                       preferred_element_type=F32)) / wsum
                    ctx_rows.append(ctx_b)
                ctx_h = jnp.concatenate(ctx_rows, axis=0)
                woh = wo_ref[pl.ds(h * DH, DH), :].astype(BF16)
                part = jnp.dot(ctx_h.astype(BF16), woh,
                               preferred_element_type=F32)
                acc[pl.ds(o, 1)] = acc[pl.ds(o, 1)] + part[None]
                return carry

            lax.fori_loop(0, HQ_PER, hstep, 0)

            if s >= 1:
                t = s - 1
                if t >= 1:
                    rs_desc(t - 1).wait_recv()
                if t >= 2:
                    rs_desc(t - 2).wait_send()
                data = acc[pl.ds(o, 1)]
                if t > 0:
                    data = data + recvbuf[pl.ds(t - 1, 1)]
                sendbuf[pl.ds(t % 2, 1)] = data
                rs_desc(t).start()

            if s < N_DEV - 1:
                ag.wait()

        rs_desc(N_DEV - 3).wait_send()
        last = rs_desc(N_DEV - 2)
        last.wait_send()
        last.wait_recv()
        out_ref[:] = acc[pl.ds(my, 1)] + recvbuf[pl.ds(N_DEV - 2, 1)]

    return pl.pallas_call(
        body,
        out_shape=jax.ShapeDtypeStruct((1, SQ_PER, DM), F32),
        in_specs=[
            pl.BlockSpec(memory_space=pltpu.VMEM),
            pl.BlockSpec(memory_space=pltpu.VMEM),
            pl.BlockSpec(memory_space=pl.ANY),
            pl.BlockSpec(memory_space=pl.ANY),
            pl.BlockSpec(memory_space=pltpu.VMEM),
        ],
        out_specs=pl.BlockSpec(memory_space=pltpu.VMEM),
        scratch_shapes=[
            pltpu.VMEM((N_DEV, SQ_PER, DM), BF16),
            pltpu.VMEM((N_DEV, SQ_PER, DM), F32),
            pltpu.VMEM((2, 3, 22 * QBLK, DH), F32),
            pltpu.VMEM((2, 3, 22 * QBLK, DH), F32),
            pltpu.VMEM((HQ_PER, 3, 22 * QBLK, DH), BF16),
            pltpu.VMEM((HQ_PER, 3, 22 * QBLK, DH), BF16),
            pltpu.VMEM((SQ_PER, DM), BF16),
            pltpu.VMEM((2, SQ_PER, DM), F32),
            pltpu.VMEM((N_DEV - 1, SQ_PER, DM), F32),
            pltpu.SemaphoreType.DMA((N_DEV - 1,)),
            pltpu.SemaphoreType.DMA((N_DEV - 1,)),
            pltpu.SemaphoreType.DMA((N_DEV - 1,)),
            pltpu.SemaphoreType.DMA((N_DEV - 1,)),
            pltpu.SemaphoreType.DMA((2, 2)),
        ],
        compiler_params=pltpu.CompilerParams(
            collective_id=0,
            vmem_limit_bytes=63 * 1024 * 1024,
        ),
    )(x, Wq, K_ext, V_ext, Wo)


# device time: 160650 ns/iter; 1.3745x vs baseline; 1.0226x over previous
import jax
import jax.numpy as jnp
from jax import lax
from jax.experimental import pallas as pl
from jax.experimental.pallas import tpu as pltpu

N_DEV = 8
SQ_PER = 256
QBLK = 64
HQ_PER = 8
DH = 128
SKV = 4096
DM = 1024
SCALE = 0.08838834764831843
F32 = jnp.float32
BF16 = jnp.bfloat16


def kernel(x, Wq, K_ext, V_ext, Wo):
    def body(x_ref, wq_ref, k_hbm, v_hbm, wo_ref, out_ref,
             xg, acc, kbuf, vbuf, kres, vres, qbuf, ctxbuf, sendbuf, recvbuf,
             ag_send, ag_recv, rs_send, rs_recv, kv_sems):
        my = lax.axis_index("i")
        left = lax.rem(my - 1 + N_DEV, N_DEV)
        right = lax.rem(my + 1, N_DEV)
        h0 = my * HQ_PER

        def kv_dma(h, slot):
            copies = []
            for c in range(3):
                for i, kbk in enumerate(range(c, SKV // QBLK, 3)):
                    copies.append(pltpu.make_async_copy(
                        k_hbm.at[0, pl.ds(kbk * QBLK, QBLK), h0 + h, :],
                        kbuf.at[slot, c, pl.ds(i * QBLK, QBLK)],
                        kv_sems.at[slot, 0]))
                    copies.append(pltpu.make_async_copy(
                        v_hbm.at[0, pl.ds(kbk * QBLK, QBLK), h0 + h, :],
                        vbuf.at[slot, c, pl.ds(i * QBLK, QBLK)],
                        kv_sems.at[slot, 1]))
            return copies

        for slot in range(2):
            for c in (1, 2):
                kbuf[slot, c, pl.ds(21 * QBLK, QBLK)] = jnp.zeros(
                    (QBLK, DH), F32)
                vbuf[slot, c, pl.ds(21 * QBLK, QBLK)] = jnp.zeros(
                    (QBLK, DH), F32)

        for cpy in kv_dma(0, 0):
            cpy.start()

        xg[pl.ds(my, 1)] = x_ref[:].astype(BF16)

        barrier = pltpu.get_barrier_semaphore()
        for nbr in (left, right):
            pl.semaphore_signal(barrier, inc=1, device_id=(nbr,),
                                device_id_type=pl.DeviceIdType.MESH)
        pl.semaphore_wait(barrier, 2)

        wqb = wq_ref[:].astype(BF16)
        wob = wo_ref[:].astype(BF16)

        def rs_desc(t):
            return pltpu.make_async_remote_copy(
                src_ref=sendbuf.at[t % 2],
                dst_ref=recvbuf.at[t],
                send_sem=rs_send.at[t],
                recv_sem=rs_recv.at[t],
                device_id=(right,),
                device_id_type=pl.DeviceIdType.MESH,
            )

        for s in range(N_DEV):
            o = lax.rem(my - s + N_DEV, N_DEV)
            if s < N_DEV - 1:
                ag = pltpu.make_async_remote_copy(
                    src_ref=xg.at[pl.ds(o, 1)],
                    dst_ref=xg.at[pl.ds(o, 1)],
                    send_sem=ag_send.at[s],
                    recv_sem=ag_recv.at[s],
                    device_id=(right,),
                    device_id_type=pl.DeviceIdType.MESH,
                )
                ag.start()

            qbuf[:] = jnp.dot(xg[pl.ds(o, 1)][0], wqb,
                              preferred_element_type=F32).astype(BF16)

            def hstep(h, carry, s=s, o=o):
                if s == 0:
                    slot = lax.rem(h, 2)
                    for cpy in kv_dma(h, slot):
                        cpy.wait()
                    @pl.when(h < HQ_PER - 1)
                    def _():
                        for cpy in kv_dma(h + 1, 1 - slot):
                            cpy.start()
                    kres[h] = kbuf[slot].astype(BF16)
                    vres[h] = vbuf[slot].astype(BF16)

                ctx_rows = []
                for lqb in range(SQ_PER // QBLK):
                    qb = o * (SQ_PER // QBLK) + lqb
                    r = lax.rem(qb, 3)
                    g = lax.rem(3 - r, 3)
                    dpos = (qb // 3) * QBLK
                    qs = qbuf[pl.ds(lqb * QBLK, QBLK),
                              pl.ds(h * DH, DH)]
                    kmain = kres[h, g]
                    smain = lax.dot_general(
                        qs, kmain, (((1,), (1,)), ((), ())),
                        preferred_element_type=F32) * SCALE
                    wmain = jnp.exp(smain)
                    kex = jnp.concatenate(
                        [kres[h, 0, pl.ds(0, QBLK)],
                         kres[h, r, pl.ds(dpos, QBLK)]], axis=0)
                    sex = lax.dot_general(
                        qs, kex, (((1,), (1,)), ((), ())),
                        preferred_element_type=F32) * SCALE
                    fac = jnp.where(r == 0, 0.0, 1.0).astype(F32)
                    wex = jnp.exp(sex) * fac
                    padc = jnp.where(g == 0, 0.0, float(QBLK)).astype(F32)
                    wsum = (jnp.sum(wmain, axis=1, keepdims=True)
                            + jnp.sum(wex, axis=1, keepdims=True) - padc)
                    vmain = vres[h, g]
                    vex = jnp.concatenate(
                        [vres[h, 0, pl.ds(0, QBLK)],
                         vres[h, r, pl.ds(dpos, QBLK)]], axis=0)
                    ctx_b = (jnp.dot(wmain.astype(BF16), vmain,
                                     preferred_element_type=F32)
                             + jnp.dot(wex.astype(BF16), vex,
                                       preferred_element_type=F32)) / wsum
                    ctx_rows.append(ctx_b)
                ctx_h = jnp.concatenate(ctx_rows, axis=0)
                ctxbuf[:, pl.ds(h * DH, DH)] = ctx_h.astype(BF16)
                return carry

            lax.fori_loop(0, HQ_PER, hstep, 0)
            acc[pl.ds(o, 1)] = jnp.dot(
                ctxbuf[:], wob, preferred_element_type=F32)[None]

            if s >= 1:
                t = s - 1
                if t >= 1:
                    rs_desc(t - 1).wait_recv()
                if t >= 2:
                    rs_desc(t - 2).wait_send()
                data = acc[pl.ds(o, 1)]
                if t > 0:
                    data = data + recvbuf[pl.ds(t - 1, 1)]
                sendbuf[pl.ds(t % 2, 1)] = data
                rs_desc(t).start()

            if s < N_DEV - 1:
                ag.wait()

        rs_desc(N_DEV - 3).wait_send()
        last = rs_desc(N_DEV - 2)
        last.wait_send()
        last.wait_recv()
        out_ref[:] = acc[pl.ds(my, 1)] + recvbuf[pl.ds(N_DEV - 2, 1)]

    return pl.pallas_call(
        body,
        out_shape=jax.ShapeDtypeStruct((1, SQ_PER, DM), F32),
        in_specs=[
            pl.BlockSpec(memory_space=pltpu.VMEM),
            pl.BlockSpec(memory_space=pltpu.VMEM),
            pl.BlockSpec(memory_space=pl.ANY),
            pl.BlockSpec(memory_space=pl.ANY),
            pl.BlockSpec(memory_space=pltpu.VMEM),
        ],
        out_specs=pl.BlockSpec(memory_space=pltpu.VMEM),
        scratch_shapes=[
            pltpu.VMEM((N_DEV, SQ_PER, DM), BF16),
            pltpu.VMEM((N_DEV, SQ_PER, DM), F32),
            pltpu.VMEM((2, 3, 22 * QBLK, DH), F32),
            pltpu.VMEM((2, 3, 22 * QBLK, DH), F32),
            pltpu.VMEM((HQ_PER, 3, 22 * QBLK, DH), BF16),
            pltpu.VMEM((HQ_PER, 3, 22 * QBLK, DH), BF16),
            pltpu.VMEM((SQ_PER, DM), BF16),
            pltpu.VMEM((SQ_PER, DM), BF16),
            pltpu.VMEM((2, SQ_PER, DM), F32),
            pltpu.VMEM((N_DEV - 1, SQ_PER, DM), F32),
            pltpu.SemaphoreType.DMA((N_DEV - 1,)),
            pltpu.SemaphoreType.DMA((N_DEV - 1,)),
            pltpu.SemaphoreType.DMA((N_DEV - 1,)),
            pltpu.SemaphoreType.DMA((N_DEV - 1,)),
            pltpu.SemaphoreType.DMA((2, 2)),
        ],
        compiler_params=pltpu.CompilerParams(
            collective_id=0,
            vmem_limit_bytes=63 * 1024 * 1024,
        ),
    )(x, Wq, K_ext, V_ext, Wo)
